# Initial kernel scaffold; baseline (speedup 1.0000x reference)
import jax
import jax.numpy as jnp
from jax.experimental import pallas as pl


def kernel(X1, X0, U, A0_w, A_ws, B_ws, C_w, C_b):
    raise NotImplementedError("write your pallas kernel here")


# speedup vs baseline: 1.1202x; 1.1202x over previous
import functools

import jax
import jax.numpy as jnp
from jax import lax
from jax.experimental import pallas as pl
from jax.experimental.pallas import tpu as pltpu
from jax.experimental.pallas import tpu_sc as plsc

N = 8192
ED = 1024
AD = 64
K = 8
ALPHA = 1.0

RBLK = 512
RM = N // RBLK

BLKM = 256
NBLK = N // BLKM + K
P = NBLK * BLKM

NW = 32
RPW = P // NW
CH = 40
ADP = 128


def _router_body(x_ref, cwt_ref, cb_ref, inds_ref, bc_ref):
    logits = jnp.dot(x_ref[...], cwt_ref[...],
                     preferred_element_type=jnp.float32) + cb_ref[...]
    lane = lax.broadcasted_iota(jnp.int32, (RBLK, 128), 1)
    logits = jnp.where(lane < K, logits, jnp.float32(-1e30))
    mx = jnp.max(logits, axis=1, keepdims=True)
    ind = jnp.min(jnp.where(logits == mx, lane, K), axis=1, keepdims=True)
    inds_ref[...] = ind
    onehot = (ind == lax.broadcasted_iota(jnp.int32, (RBLK, K), 1))
    bc_ref[...] = jnp.sum(onehot.astype(jnp.float32), axis=0,
                          keepdims=True).reshape(1, 1, K)


def _slot_body(inds_ref, base_ref, dst_ref):
    ind = inds_ref[...]
    onehot = (ind == lax.broadcasted_iota(jnp.int32, (RBLK, K), 1)
              ).astype(jnp.float32)
    row = lax.broadcasted_iota(jnp.int32, (RBLK, RBLK), 0)
    col = lax.broadcasted_iota(jnp.int32, (RBLK, RBLK), 1)
    tri = (col < row).astype(jnp.float32)
    rank = jnp.dot(tri, onehot, preferred_element_type=jnp.float32)
    slot = jnp.sum(onehot * (rank + base_ref[0]), axis=1, keepdims=True)
    dst_ref[...] = slot.astype(jnp.int32)


def _gather_body(dst_hbm, x0_hbm, x1_hbm, u_hbm,
                 xs_out, x1s_out, us_out,
                 g_spmem, dst_v, g_vmem, idx_v, xbuf, ubuf, sem):
    cid = lax.axis_index("c")
    sid = lax.axis_index("s")

    @pl.when(sid == 0)
    def _phase_a():
        pltpu.sync_copy(dst_hbm, dst_v)

        def init_body(i, c):
            g_vmem[pl.ds(i * 16, 16)] = jnp.zeros((16,), jnp.int32)
            return c
        lax.fori_loop(0, P // 16, init_body, 0)

        def scat_body(i, c):
            idx16 = dst_v[pl.ds(i * 16, 16)]
            vals = lax.iota(jnp.int32, 16) + i * 16
            plsc.store_scatter(g_vmem, [idx16], vals)
            return c
        lax.fori_loop(0, N // 16, scat_body, 0)
        pltpu.sync_copy(g_vmem, g_spmem)

    plsc.subcore_barrier()

    wid = sid * 2 + cid
    base = wid * RPW
    pltpu.sync_copy(g_spmem.at[pl.ds(base, RPW)], idx_v)

    def chunk(j, c):
        off = pl.multiple_of(j * CH, CH)
        idx = idx_v.at[pl.ds(off, CH)]
        pltpu.async_copy(x0_hbm.at[idx], xbuf, sem).wait()
        pltpu.sync_copy(xbuf, xs_out.at[pl.ds(base + off, CH)])
        pltpu.async_copy(x1_hbm.at[idx], xbuf, sem).wait()
        pltpu.sync_copy(xbuf, x1s_out.at[pl.ds(base + off, CH)])
        pltpu.async_copy(u_hbm.at[idx], ubuf, sem).wait()
        pltpu.sync_copy(ubuf, us_out.at[pl.ds(base + off, CH)])
        return c
    lax.fori_loop(0, RPW // CH, chunk, 0)


def _group_mm_body(be_ref, bv_ref, xs_ref, x1s_ref, us_ref, at_ref, bt_ref,
                   out_ref):
    m = pl.program_id(0)
    pred = jnp.dot(xs_ref[...], at_ref[0],
                   preferred_element_type=jnp.float32)
    pred = pred + jnp.dot(us_ref[...], bt_ref[0],
                          preferred_element_type=jnp.float32)
    diff = x1s_ref[...] - pred
    v = bv_ref[m]
    rmask = lax.broadcasted_iota(jnp.int32, (BLKM, ED), 0) < v
    d = jnp.where(rmask, diff, jnp.float32(0.0))
    part = jnp.sum(d * d)

    @pl.when(m == 0)
    def _():
        out_ref[0, 0] = jnp.float32(0.0)

    out_ref[0, 0] += part

    @pl.when(m == NBLK - 1)
    def _():
        out_ref[0, 0] = out_ref[0, 0] * jnp.float32(ALPHA / (ED * N))


def kernel(X1, X0, U, A0_w, A_ws, B_ws, C_w, C_b):
    f32 = jnp.float32

    cwt = jnp.zeros((ED, 128), f32).at[:, :K].set(C_w.T)
    cb = jnp.zeros((1, 128), f32).at[0, :K].set(C_b)
    inds, bcounts = pl.pallas_call(
        _router_body,
        grid=(RM,),
        in_specs=[
            pl.BlockSpec((RBLK, ED), lambda m: (m, 0)),
            pl.BlockSpec((ED, 128), lambda m: (0, 0)),
            pl.BlockSpec((1, 128), lambda m: (0, 0)),
        ],
        out_specs=[
            pl.BlockSpec((RBLK, 1), lambda m: (m, 0)),
            pl.BlockSpec((1, 1, K), lambda m: (m, 0, 0)),
        ],
        out_shape=[
            jax.ShapeDtypeStruct((N, 1), jnp.int32),
            jax.ShapeDtypeStruct((RM, 1, K), f32),
        ],
    )(X0, cwt, cb)

    bc = bcounts.reshape(RM, K)
    counts = jnp.sum(bc, axis=0).astype(jnp.int32)
    padded = ((counts + BLKM - 1) // BLKM) * BLKM
    starts = jnp.concatenate(
        [jnp.zeros((1,), jnp.int32), jnp.cumsum(padded)[:-1].astype(jnp.int32)])
    excl = jnp.concatenate(
        [jnp.zeros((1, K), f32), jnp.cumsum(bc, axis=0)[:-1]], axis=0)
    base_tab = (starts.astype(f32)[None, :] + excl).reshape(RM, 1, K)

    nblocks_e = padded // BLKM
    bstart_e = starts // BLKM
    b = jnp.arange(NBLK, dtype=jnp.int32)
    in_e = (b[:, None] >= bstart_e[None, :]) & (
        b[:, None] < (bstart_e + nblocks_e)[None, :])
    blk_expert = jnp.where(jnp.any(in_e, axis=1),
                           jnp.argmax(in_e, axis=1).astype(jnp.int32),
                           jnp.int32(K - 1))
    vexp = jnp.clip(counts[blk_expert] - (b - bstart_e[blk_expert]) * BLKM,
                    0, BLKM)
    blk_valid = jnp.where(jnp.any(in_e, axis=1), vexp, 0).astype(jnp.int32)

    dst = pl.pallas_call(
        _slot_body,
        grid=(RM,),
        in_specs=[
            pl.BlockSpec((RBLK, 1), lambda m: (m, 0)),
            pl.BlockSpec((1, 1, K), lambda m: (m, 0, 0)),
        ],
        out_specs=pl.BlockSpec((RBLK, 1), lambda m: (m, 0)),
        out_shape=jax.ShapeDtypeStruct((N, 1), jnp.int32),
    )(inds, base_tab)

    mesh = plsc.VectorSubcoreMesh(core_axis_name="c", subcore_axis_name="s")
    gather = pl.kernel(
        _gather_body,
        out_type=[
            jax.ShapeDtypeStruct((P, ED), f32),
            jax.ShapeDtypeStruct((P, ED), f32),
            jax.ShapeDtypeStruct((P, ADP), f32),
        ],
        mesh=mesh,
        compiler_params=pltpu.CompilerParams(needs_layout_passes=False),
        scratch_types=[
            pltpu.VMEM_SHARED((P,), jnp.int32),
            pltpu.VMEM((N,), jnp.int32),
            pltpu.VMEM((P,), jnp.int32),
            pltpu.VMEM((RPW,), jnp.int32),
            pltpu.VMEM((CH, ED), f32),
            pltpu.VMEM((CH, ADP), f32),
            pltpu.SemaphoreType.DMA,
        ],
    )
    U_pad = jnp.zeros((N, ADP), f32).at[:, :AD].set(U)
    Xs, X1s, Us = gather(dst.reshape(N), X0, X1, U_pad)

    AT = jnp.concatenate([A0_w.T[None], jnp.transpose(A_ws, (0, 2, 1))], axis=0)
    Bt = jnp.concatenate([jnp.eye(AD, ED, dtype=f32)[None],
                          jnp.transpose(B_ws, (0, 2, 1))], axis=0)
    Bt = jnp.zeros((K, ADP, ED), f32).at[:, :AD, :].set(Bt)

    grid_spec = pltpu.PrefetchScalarGridSpec(
        num_scalar_prefetch=2,
        grid=(NBLK,),
        in_specs=[
            pl.BlockSpec((BLKM, ED), lambda m, be, bv: (m, 0)),
            pl.BlockSpec((BLKM, ED), lambda m, be, bv: (m, 0)),
            pl.BlockSpec((BLKM, ADP), lambda m, be, bv: (m, 0)),
            pl.BlockSpec((1, ED, ED), lambda m, be, bv: (be[m], 0, 0)),
            pl.BlockSpec((1, ADP, ED), lambda m, be, bv: (be[m], 0, 0)),
        ],
        out_specs=pl.BlockSpec(memory_space=pltpu.MemorySpace.SMEM),
    )
    out = pl.pallas_call(
        _group_mm_body,
        grid_spec=grid_spec,
        out_shape=jax.ShapeDtypeStruct((1, 1), f32),
    )(blk_expert, blk_valid, Xs, X1s, Us, AT, Bt)

    return out[0, 0]
